# baseline (speedup 1.0000x reference)
import functools

import jax
import jax.numpy as jnp
from jax import lax
from jax.experimental import pallas as pl
from jax.experimental.pallas import tpu as pltpu
from jax.experimental.pallas import tpu_sc as plsc

_NUM_CORES = 2
_NUM_SUBCORES = 16
_NUM_WORKERS = _NUM_CORES * _NUM_SUBCORES
_PCHUNK = 8
_NBUF = 4
_LANES = 16


def _emb_lookup(idx_flat, wte, wpe, b):
    n = idx_flat.shape[0]
    _, d = wte.shape
    s = wpe.shape[0]
    pos_w = s // _NUM_WORKERS
    per_w = n // _NUM_WORKERS
    n_super = pos_w // _PCHUNK
    rows = b * _PCHUNK
    slices_per_row = d // _LANES
    mesh = plsc.VectorSubcoreMesh(core_axis_name="c", subcore_axis_name="s")

    half = n_super // 2
    scratch = (
        [pltpu.VMEM((per_w,), jnp.int32)]
        + [pltpu.VMEM((half * _PCHUNK, d), jnp.float32)]
        + [pltpu.VMEM((rows, d), jnp.float32)] * _NBUF
        + [pltpu.SemaphoreType.DMA] * (2 * _NBUF + 1)
    )

    @functools.partial(
        pl.kernel,
        out_type=jax.ShapeDtypeStruct((n, d), jnp.float32),
        mesh=mesh,
        scratch_types=scratch,
    )
    def body(idx_hbm, wte_hbm, wpe_hbm, out_hbm, idx_v, wpe_v, *rest):
        bufs = rest[:_NBUF]
        gsem = rest[_NBUF:2 * _NBUF]
        osem = rest[2 * _NBUF:3 * _NBUF]
        wsem = rest[3 * _NBUF]

        wid = lax.axis_index("s") * _NUM_CORES + lax.axis_index("c")
        p_base = wid * pos_w

        idx_h = [
            pltpu.async_copy(
                idx_hbm.at[pl.ds(bi * s + p_base, pos_w)],
                idx_v.at[pl.ds(bi * pos_w, pos_w)],
                wsem,
            )
            for bi in range(b)
        ]
        for h in idx_h:
            h.wait()

        def issue(t):
            buf = bufs[t % _NBUF]
            sem = gsem[t % _NBUF]
            return [
                pltpu.async_copy(
                    wte_hbm.at[
                        idx_v.at[pl.ds(bi * pos_w + t * _PCHUNK, _PCHUNK)]
                    ],
                    buf.at[pl.ds(bi * _PCHUNK, _PCHUNK)],
                    sem,
                )
                for bi in range(b)
            ]

        inflight = {t: issue(t) for t in range(min(2, n_super))}
        wpe_h = pltpu.async_copy(
            wpe_hbm.at[pl.ds(p_base, half * _PCHUNK)], wpe_v, wsem
        )

        out_h = {}
        wpe_h.wait()
        for t in range(n_super):
            if t == half:
                pltpu.sync_copy(
                    wpe_hbm.at[pl.ds(p_base + half * _PCHUNK, half * _PCHUNK)],
                    wpe_v,
                )
            for h in inflight.pop(t):
                h.wait()
            buf = bufs[t % _NBUF]

            def add_row(r, carry):
                for j in range(slices_per_row):
                    x = wpe_v[
                        (t % half) * _PCHUNK + r, pl.ds(j * _LANES, _LANES)
                    ]
                    for bi in range(b):
                        sl = (bi * _PCHUNK + r, pl.ds(j * _LANES, _LANES))
                        buf[sl] = buf[sl] + x
                return carry

            lax.fori_loop(0, _PCHUNK, add_row, 0)
            out_h[t] = [
                pltpu.async_copy(
                    buf.at[pl.ds(bi * _PCHUNK, _PCHUNK)],
                    out_hbm.at[pl.ds(bi * s + p_base + t * _PCHUNK, _PCHUNK)],
                    osem[t % _NBUF],
                )
                for bi in range(b)
            ]
            if t + 2 < n_super:
                if t + 2 - _NBUF in out_h:
                    for h in out_h.pop(t + 2 - _NBUF):
                        h.wait()
                inflight[t + 2] = issue(t + 2)
        for t in sorted(out_h):
            for h in out_h.pop(t):
                h.wait()

    return body(idx_flat, wte, wpe)


def kernel(idx, wte, wpe):
    b, s = idx.shape
    d = wte.shape[1]
    idx_flat = idx.reshape(b * s).astype(jnp.int32)
    out = _emb_lookup(idx_flat, wte, wpe, b)
    return out.reshape(b, s, d)


# speedup vs baseline: 1.0297x; 1.0297x over previous
import functools

import jax
import jax.numpy as jnp
from jax import lax
from jax.experimental import pallas as pl
from jax.experimental.pallas import tpu as pltpu
from jax.experimental.pallas import tpu_sc as plsc

_NUM_CORES = 2
_NUM_SUBCORES = 16
_NUM_WORKERS = _NUM_CORES * _NUM_SUBCORES
_PCHUNK = 8
_NBUF = 3
_LANES = 16


def _emb_lookup(idx_flat, wte, wpe, b):
    n = idx_flat.shape[0]
    _, d = wte.shape
    s = wpe.shape[0]
    pos_w = s // _NUM_WORKERS
    per_w = n // _NUM_WORKERS
    n_super = pos_w // _PCHUNK
    rows = b * _PCHUNK
    slices_per_row = d // _LANES
    mesh = plsc.VectorSubcoreMesh(core_axis_name="c", subcore_axis_name="s")

    scratch = (
        [pltpu.VMEM((per_w,), jnp.int32)]
        + [pltpu.VMEM((pos_w, d), jnp.float32)]
        + [pltpu.VMEM((rows, d), jnp.float32)] * _NBUF
        + [pltpu.SemaphoreType.DMA] * (2 * _NBUF + 1)
    )

    @functools.partial(
        pl.kernel,
        out_type=jax.ShapeDtypeStruct((n, d), jnp.float32),
        mesh=mesh,
        scratch_types=scratch,
    )
    def body(idx_hbm, wte_hbm, wpe_hbm, out_hbm, idx_v, wpe_v, *rest):
        bufs = rest[:_NBUF]
        gsem = rest[_NBUF:2 * _NBUF]
        osem = rest[2 * _NBUF:3 * _NBUF]
        wsem = rest[3 * _NBUF]

        wid = lax.axis_index("s") * _NUM_CORES + lax.axis_index("c")
        p_base = wid * pos_w

        idx_h = [
            pltpu.async_copy(
                idx_hbm.at[pl.ds(bi * s + p_base, pos_w)],
                idx_v.at[pl.ds(bi * pos_w, pos_w)],
                wsem,
            )
            for bi in range(b)
        ]
        for h in idx_h:
            h.wait()

        def issue(t):
            buf = bufs[t % _NBUF]
            sem = gsem[t % _NBUF]
            return [
                pltpu.async_copy(
                    wte_hbm.at[
                        idx_v.at[pl.ds(bi * pos_w + t * _PCHUNK, _PCHUNK)]
                    ],
                    buf.at[pl.ds(bi * _PCHUNK, _PCHUNK)],
                    sem,
                )
                for bi in range(b)
            ]

        inflight = {t: issue(t) for t in range(min(2, n_super))}
        wpe_h = pltpu.async_copy(wpe_hbm.at[pl.ds(p_base, pos_w)], wpe_v, wsem)

        out_h = {}
        wpe_h.wait()
        for t in range(n_super):
            for h in inflight.pop(t):
                h.wait()
            buf = bufs[t % _NBUF]

            def add_row(r, carry):
                for j in range(slices_per_row):
                    x = wpe_v[t * _PCHUNK + r, pl.ds(j * _LANES, _LANES)]
                    for bi in range(b):
                        sl = (bi * _PCHUNK + r, pl.ds(j * _LANES, _LANES))
                        buf[sl] = buf[sl] + x
                return carry

            lax.fori_loop(0, _PCHUNK, add_row, 0)
            out_h[t] = [
                pltpu.async_copy(
                    buf.at[pl.ds(bi * _PCHUNK, _PCHUNK)],
                    out_hbm.at[pl.ds(bi * s + p_base + t * _PCHUNK, _PCHUNK)],
                    osem[t % _NBUF],
                )
                for bi in range(b)
            ]
            if t + 2 < n_super:
                if t + 2 - _NBUF in out_h:
                    for h in out_h.pop(t + 2 - _NBUF):
                        h.wait()
                inflight[t + 2] = issue(t + 2)
        for t in sorted(out_h):
            for h in out_h.pop(t):
                h.wait()

    return body(idx_flat, wte, wpe)


def kernel(idx, wte, wpe):
    b, s = idx.shape
    d = wte.shape[1]
    idx_flat = idx.reshape(b * s).astype(jnp.int32)
    out = _emb_lookup(idx_flat, wte, wpe, b)
    return out.reshape(b, s, d)
